# baseline (speedup 1.0000x reference)
import jax
import jax.numpy as jnp
from jax.experimental import pallas as pl

_L_BLK = 512


def _transpose_body(v_ref, o_ref):
    o_ref[...] = jnp.transpose(v_ref[...], (0, 2, 1, 3))


def kernel(queries, keys, values):
    b, l, h, d = values.shape
    return pl.pallas_call(
        _transpose_body,
        grid=(b, l // _L_BLK),
        in_specs=[pl.BlockSpec((1, _L_BLK, h, d), lambda i, j: (i, j, 0, 0))],
        out_specs=pl.BlockSpec((1, h, _L_BLK, d), lambda i, j: (i, 0, j, 0)),
        out_shape=jax.ShapeDtypeStruct((b, h, l, d), values.dtype),
    )(values)


# speedup vs baseline: 1.4621x; 1.4621x over previous
import jax
import jax.numpy as jnp
from jax.experimental import pallas as pl

_L_BLK = 1024


def _split_body(v_ref, o_ref):
    v = v_ref[0]
    h = o_ref.shape[1]
    d = o_ref.shape[3]
    for i in range(h):
        o_ref[0, i] = v[:, i * d:(i + 1) * d]


def kernel(queries, keys, values):
    b, l, h, d = values.shape
    v2 = values.reshape(b, l, h * d)
    return pl.pallas_call(
        _split_body,
        grid=(b, l // _L_BLK),
        in_specs=[pl.BlockSpec((1, _L_BLK, h * d), lambda i, k: (i, k, 0))],
        out_specs=pl.BlockSpec((1, h, _L_BLK, d), lambda i, k: (i, 0, k, 0)),
        out_shape=jax.ShapeDtypeStruct((b, h, l, d), values.dtype),
    )(v2)
